# baseline (speedup 1.0000x reference)
import jax
import jax.numpy as jnp
from jax import lax
from jax.experimental import pallas as pl

_G = 10
_K = 500
_INTP = 13


def _lane_scalar(vec, lane_idx, lane_iota):
    return jnp.sum(jnp.where(lane_iota == lane_idx, vec, 0.0))


def _backscatter_body(x_ref, d_ref, bc_ref, enb_ref, o_ref):
    B, C, R, L = x_ref.shape
    N = R * L
    f32 = jnp.float32
    i32 = jnp.int32
    idx_bits = int(N - 1).bit_length()
    BIGI = jnp.int32(0x7FFFFFFF)

    lane_r = lax.broadcasted_iota(i32, (1, L), 1)
    row10 = lax.broadcasted_iota(i32, (_G, L), 0)
    lane10 = lax.broadcasted_iota(i32, (_G, L), 1)
    diag10 = row10 == lane10

    def sra31(x):
        return lax.shift_right_arithmetic(x, jnp.int32(31))

    def ltm(a, b):
        return sra31(a - b)

    def eqm(a, b):
        x = a - b
        return ~sra31(x | (jnp.int32(0) - x))

    def all_lanes(x, op):
        s = 1
        while s < L:
            x = op(x, jnp.roll(x, s, axis=1))
            s *= 2
        return x

    def pack_from_rows(rows_uniform, zero):
        return jnp.sum(
            jnp.where(diag10, rows_uniform, zero), axis=0, keepdims=True
        )

    def rows_from_pack(packv, zero):
        d = jnp.where(diag10, jnp.broadcast_to(packv, (_G, L)), zero)
        return all_lanes(d, jnp.add)

    def counts_pack_neg(parts):
        rows = jnp.concatenate(parts, axis=0)
        return -pack_from_rows(all_lanes(rows, jnp.add), jnp.int32(0))

    dall = d_ref[...]
    dmin = jnp.min(dall)
    dmax = jnp.max(dall)

    def two_sum(a, b):
        s = a + b
        v = s - a
        e = (a - (s - v)) + (b - v)
        return s, e

    def split(a):
        c = a * f32(4097.0)
        hi = c - (c - a)
        return hi, a - hi

    def two_prod(a, b):
        p = a * b
        ah, al = split(a)
        bh, bl = split(b)
        e = ((ah * bh - p) + ah * bl + al * bh) + al * bl
        return p, e

    g = f32(_G)
    dh, dl = two_sum(dmax, -dmin)
    q1 = dh / g
    p, pe = two_prod(q1, g)
    t, te = two_sum(dh, -p)
    r = t + ((te - pe) + dl)
    q2 = r / g
    s_hi, s_lo = two_sum(q1, q2)
    idxv = lane_r.astype(f32)
    ph, pe2 = two_prod(jnp.full((1, L), s_hi), idxv)
    plo = pe2 + s_lo * idxv
    th, te2 = two_sum(ph, jnp.full((1, L), dmin))
    iv = th + (te2 + plo)
    iv = jnp.where(lane_r == 0, f32(0.0), iv)
    iv = jnp.where(lane_r == _G, dmax, iv)
    ivb = lax.bitcast_convert_type(iv, i32)
    ivbits = [
        jnp.sum(jnp.where(lane_r == j, ivb, 0)) for j in range(_G + 1)
    ]

    gmaps = []
    tot_rows = jnp.zeros((_G + 1, L), i32)
    for b in range(B):
        dbits = lax.bitcast_convert_type(d_ref[b], i32)
        gtn = jnp.zeros((R, L), i32)
        parts = []
        for j in range(_G + 1):
            m = ltm(ivbits[j], dbits)
            gtn = gtn + m
            parts.append(jnp.sum(m, axis=0, keepdims=True))
        gmaps.append(jnp.int32(-1) - gtn)
        tot_rows = tot_rows + jnp.concatenate(parts, axis=0)
    tot_rows = all_lanes(tot_rows, jnp.add)
    cnt_rows10 = tot_rows[1 : _G + 1, :] - tot_rows[0:_G, :]
    cnt_pack = pack_from_rows(cnt_rows10, jnp.int32(0))
    numpix = cnt_pack.astype(f32) / f32(B)
    kpack = jnp.minimum(jnp.ceil(numpix * f32(0.01)), f32(_K)).astype(i32)
    kpackf = kpack.astype(f32)
    k_rows = rows_from_pack(kpack, jnp.int32(0))
    kpos_rows = sra31(jnp.int32(0) - k_rows)

    lgrows = [jnp.log(enb_ref[c : c + 1, :]) for c in range(C)]
    bcrows = [bc_ref[c : c + 1, :] for c in range(C)]

    pix_idx = (
        lax.broadcasted_iota(i32, (R, L), 0) * L
        + lax.broadcasted_iota(i32, (R, L), 1)
    )

    num_part = jnp.zeros((1, L), f32)
    den_part = jnp.zeros((1, L), f32)

    for b in range(B):
        db = d_ref[b]
        bright = (x_ref[b, 0] + x_ref[b, 1] + x_ref[b, 2]) / f32(C)
        bbits = lax.bitcast_convert_type(bright, i32)
        mbits = lax.bitcast_convert_type(bright * f32(1000.0), i32)
        gmap = gmaps[b]
        v = [jnp.where(gmap == i, bbits, mbits) for i in range(_G)]

        def group_counts(rows_bound):
            parts = [
                jnp.sum(
                    ltm(v[i], rows_bound[i : i + 1, :]), axis=0, keepdims=True
                )
                for i in range(_G)
            ]
            return counts_pack_neg(parts)

        tgt = kpackf - f32(0.5)

        def interp_body(it, carry):
            lob, hib, clot, chit, cloe, chie, last = carry
            conv = (
                (clot == kpack - 1)
                | (chit == kpack)
                | (hib - lob == 1)
                | (kpack == 0)
            )
            lo_f = lax.bitcast_convert_type(lob, f32)
            hi_f = lax.bitcast_convert_type(hib, f32)
            frac = (tgt - cloe) / jnp.maximum(chie - cloe, f32(1e-9))
            frac = jnp.clip(frac, f32(0.0), f32(1.0))
            piv_f = lo_f + (hi_f - lo_f) * frac
            piv_b = lax.bitcast_convert_type(piv_f, i32)
            piv_b = jnp.clip(piv_b, lob + 1, jnp.maximum(hib - 1, lob + 1))
            piv_rows = rows_from_pack(piv_b, jnp.int32(0))
            c = group_counts(piv_rows)
            cf = c.astype(f32)
            less = c < kpack
            lo_upd = (~conv) & less
            hi_upd = (~conv) & (~less)
            new_cloe = jnp.where(
                lo_upd,
                cf,
                jnp.where(
                    hi_upd & (last == 1), tgt - (tgt - cloe) * f32(0.5), cloe
                ),
            )
            new_chie = jnp.where(
                hi_upd,
                cf,
                jnp.where(
                    lo_upd & (last == -1), tgt + (chie - tgt) * f32(0.5), chie
                ),
            )
            return (
                jnp.where(lo_upd, piv_b, lob),
                jnp.where(hi_upd, piv_b, hib),
                jnp.where(lo_upd, c, clot),
                jnp.where(hi_upd, c, chit),
                new_cloe,
                new_chie,
                jnp.where(lo_upd, jnp.int32(-1), jnp.where(hi_upd, jnp.int32(1), last)),
            )

        init = (
            jnp.zeros((1, L), i32),
            jnp.full((1, L), jnp.int32(0x44800000)),
            jnp.zeros((1, L), i32),
            jnp.full((1, L), jnp.int32(N)),
            jnp.zeros((1, L), f32),
            jnp.full((1, L), f32(N)),
            jnp.zeros((1, L), i32),
        )
        lob, hib, clot, chit, _, _, _ = lax.fori_loop(0, _INTP, interp_body, init)

        conv_min = (clot == kpack - 1) & (kpack > 0)
        conv_max = chit == kpack
        conv_w1 = hib - lob == 1
        ok = conv_min | conv_max | conv_w1 | (kpack == 0)
        all_ok = jnp.sum(jnp.where(ok, 0, 1)) == 0

        def ts_from_brackets(_):
            lob_rows = rows_from_pack(lob, jnp.int32(0))
            hib_rows = rows_from_pack(hib, jnp.int32(0))
            mn_parts, mx_parts = [], []
            for i in range(_G):
                inm = ltm(v[i], hib_rows[i : i + 1, :]) & ~ltm(
                    v[i], lob_rows[i : i + 1, :]
                )
                mn_parts.append(
                    jnp.min((v[i] & inm) | (BIGI & ~inm), axis=0, keepdims=True)
                )
                mx_parts.append(
                    jnp.max(v[i] | ~inm, axis=0, keepdims=True)
                )
            mn_rows = all_lanes(jnp.concatenate(mn_parts, axis=0), jnp.minimum)
            mx_rows = all_lanes(jnp.concatenate(mx_parts, axis=0), jnp.maximum)
            mnp = pack_from_rows(mn_rows, jnp.int32(0))
            mxp = pack_from_rows(mx_rows, jnp.int32(0))
            return jnp.where(conv_min, mnp, jnp.where(conv_max, mxp, lob))

        def ts_bitwise(_):
            def p1_body(it, tsp):
                bitval = jnp.left_shift(jnp.int32(1), 30 - it)
                cand = tsp + bitval
                c = group_counts(rows_from_pack(cand, jnp.int32(0)))
                return jnp.where(c < kpack, cand, tsp)

            return lax.fori_loop(0, 31, p1_body, jnp.zeros((1, L), i32))

        ts_pack = lax.cond(all_ok, ts_from_brackets, ts_bitwise, None)
        ts_rows = rows_from_pack(ts_pack, jnp.int32(0))

        c1_pack = group_counts(ts_rows)
        tie_m = [eqm(v[i], ts_rows[i : i + 1, :]) for i in range(_G)]

        nm = (kpack - c1_pack >= 2) & (kpack > 0)
        need_multi = jnp.sum(jnp.where(nm, 1, 0)) > 0

        def p2_easy(_):
            parts = [
                jnp.min(
                    (pix_idx & tie_m[i]) | (BIGI & ~tie_m[i]),
                    axis=0,
                    keepdims=True,
                )
                for i in range(_G)
            ]
            rows = all_lanes(jnp.concatenate(parts, axis=0), jnp.minimum)
            return pack_from_rows(rows, jnp.int32(0))

        def p2_hard(_):
            def p2_body(it, jsp):
                bitval = jnp.left_shift(jnp.int32(1), idx_bits - 1 - it)
                cand = jsp + bitval
                cand_rows = rows_from_pack(cand, jnp.int32(0))
                parts = [
                    jnp.sum(
                        tie_m[i] & ltm(pix_idx, cand_rows[i : i + 1, :]),
                        axis=0,
                        keepdims=True,
                    )
                    for i in range(_G)
                ]
                c2 = counts_pack_neg(parts)
                return jnp.where(c1_pack + c2 < kpack, cand, jsp)

            return lax.fori_loop(
                0, idx_bits, p2_body, jnp.zeros((1, L), i32)
            )

        js_pack = lax.cond(need_multi, p2_hard, p2_easy, None)
        js_rows = rows_from_pack(js_pack + 1, jnp.int32(0))

        selm = jnp.zeros((R, L), i32)
        for i in range(_G):
            sm = ltm(v[i], ts_rows[i : i + 1, :]) | (
                tie_m[i] & ltm(pix_idx, js_rows[i : i + 1, :])
            )
            selm = selm | (sm & kpos_rows[i : i + 1, :])

        rsum = jnp.zeros((R, L), f32)
        for c in range(C):
            tgtc = bcrows[c] * (f32(1.0) - jnp.exp(db * lgrows[c]))
            rsum = rsum + jnp.abs(x_ref[b, c] - tgtc)
        self_f = lax.bitcast_convert_type(
            lax.bitcast_convert_type(rsum, i32) & selm, f32
        )
        num_part = num_part + jnp.sum(self_f, axis=0, keepdims=True)
        den_part = den_part + jnp.sum(selm, axis=0, keepdims=True).astype(f32)

    num_acc = jnp.sum(num_part)
    den_acc = -jnp.sum(den_part)
    o_ref[...] = (num_acc / den_acc) * jnp.ones((1, 1), f32)


def kernel(x, depth, B_c, exp_negative_beta_b):
    B, C, H, W = x.shape
    N = H * W
    L = 128
    R = N // L
    xr = x.reshape(B, C, R, L)
    dr = depth.reshape(B, R, L)
    bc = jnp.zeros((8, L), jnp.float32).at[:C].set(
        jnp.broadcast_to(B_c.reshape(C, 1), (C, L))
    )
    enb = jnp.ones((8, L), jnp.float32).at[:C].set(
        jnp.broadcast_to(exp_negative_beta_b.reshape(C, 1), (C, L))
    )
    out = pl.pallas_call(
        _backscatter_body,
        out_shape=jax.ShapeDtypeStruct((1, 1), jnp.float32),
    )(xr, dr, bc, enb)
    return out[0, 0]


# speedup vs baseline: 1.1479x; 1.1479x over previous
import jax
import jax.numpy as jnp
from jax import lax
from jax.experimental import pallas as pl

_GROUPS = 10
_K = 500


def _lane_scalar(vec, lane_idx, lane_iota):
    return jnp.sum(jnp.where(lane_iota == lane_idx, vec, 0.0))


def _backscatter_body(x_ref, d_ref, bc_ref, enb_ref, o_ref):
    B, C, R, L = x_ref.shape
    N = R * L
    f32 = jnp.float32
    i32 = jnp.int32
    idx_bits = int(N - 1).bit_length()

    dall = d_ref[...]
    dmin = jnp.min(dall)
    dmax = jnp.max(dall)

    def two_sum(a, b):
        s = a + b
        v = s - a
        e = (a - (s - v)) + (b - v)
        return s, e

    def split(a):
        c = a * f32(4097.0)
        hi = c - (c - a)
        return hi, a - hi

    def two_prod(a, b):
        p = a * b
        ah, al = split(a)
        bh, bl = split(b)
        e = ((ah * bh - p) + ah * bl + al * bh) + al * bl
        return p, e

    lane = lax.broadcasted_iota(i32, (1, L), 1)
    g = f32(_GROUPS)
    dh, dl = two_sum(dmax, -dmin)
    q1 = dh / g
    p, pe = two_prod(q1, g)
    t, te = two_sum(dh, -p)
    r = t + ((te - pe) + dl)
    q2 = r / g
    s_hi, s_lo = two_sum(q1, q2)
    idxv = lane.astype(f32)
    ph, pe2 = two_prod(jnp.full((1, L), s_hi), idxv)
    plo = pe2 + s_lo * idxv
    th, te2 = two_sum(ph, jnp.full((1, L), dmin))
    iv = th + (te2 + plo)
    iv = jnp.where(lane == 0, f32(0.0), iv)
    iv = jnp.where(lane == _GROUPS, dmax, iv)
    intervals = [_lane_scalar(iv, j, lane) for j in range(_GROUPS + 1)]

    cnts = [jnp.int32(0) for _ in range(_GROUPS)]
    gmaps = []
    for b in range(B):
        db = d_ref[b]
        gt = jnp.zeros((R, L), i32)
        for j in range(_GROUPS + 1):
            gt = gt + (db > intervals[j]).astype(i32)
        gmap = gt - 1
        gmaps.append(gmap)
        for i in range(_GROUPS):
            cnts[i] = cnts[i] + jnp.sum((gmap == i).astype(i32))
    ks = []
    for i in range(_GROUPS):
        numpix = cnts[i].astype(f32) / f32(B)
        kf = jnp.minimum(jnp.ceil(numpix * f32(0.01)), f32(_K))
        ks.append(kf.astype(i32))

    lgrows = [jnp.log(enb_ref[c : c + 1, :]) for c in range(C)]
    bcrows = [bc_ref[c : c + 1, :] for c in range(C)]

    pix_idx = (
        lax.broadcasted_iota(i32, (R, L), 0) * L
        + lax.broadcasted_iota(i32, (R, L), 1)
    )

    num_acc = f32(0.0)
    den_acc = f32(0.0)

    for b in range(B):
        db = d_ref[b]
        bright = (x_ref[b, 0] + x_ref[b, 1] + x_ref[b, 2]) / f32(C)
        bbits = lax.bitcast_convert_type(bright, i32)
        mbits = lax.bitcast_convert_type(bright * f32(1000.0), i32)
        gmap = gmaps[b]
        v = [
            jnp.where(gmap == i, bbits, mbits) for i in range(_GROUPS)
        ]

        def p1_body(it, ts):
            bitval = jnp.left_shift(jnp.int32(1), 30 - it)
            new = []
            for i in range(_GROUPS):
                cand = ts[i] + bitval
                cnt = jnp.sum((v[i] < cand).astype(i32))
                new.append(jnp.where(cnt < ks[i], cand, ts[i]))
            return tuple(new)

        ts = lax.fori_loop(
            0, 31, p1_body, tuple(jnp.int32(0) for _ in range(_GROUPS))
        )

        c1 = [jnp.sum((v[i] < ts[i]).astype(i32)) for i in range(_GROUPS)]
        tie = [v[i] == ts[i] for i in range(_GROUPS)]

        need_multi = jnp.bool_(False)
        for i in range(_GROUPS):
            need_multi = need_multi | ((ks[i] - c1[i] >= 2) & (ks[i] > 0))

        def p2_easy(_):
            return tuple(
                jnp.min(jnp.where(tie[i], pix_idx, jnp.int32(1 << 30)))
                for i in range(_GROUPS)
            )

        def p2_hard(_):
            def p2_body(it, js):
                bitval = jnp.left_shift(jnp.int32(1), idx_bits - 1 - it)
                new = []
                for i in range(_GROUPS):
                    cand = js[i] + bitval
                    cnt2 = jnp.sum((tie[i] & (pix_idx < cand)).astype(i32))
                    new.append(jnp.where(c1[i] + cnt2 < ks[i], cand, js[i]))
                return tuple(new)

            return lax.fori_loop(
                0, idx_bits, p2_body,
                tuple(jnp.int32(0) for _ in range(_GROUPS)),
            )

        js = lax.cond(need_multi, p2_hard, p2_easy, None)

        sel = jnp.zeros((R, L), jnp.bool_)
        for i in range(_GROUPS):
            si = (v[i] < ts[i]) | (tie[i] & (pix_idx <= js[i]))
            si = si & (ks[i] > 0)
            sel = sel | si

        rsum = jnp.zeros((R, L), f32)
        for c in range(C):
            tgt = bcrows[c] * (f32(1.0) - jnp.exp(db * lgrows[c]))
            rsum = rsum + jnp.abs(x_ref[b, c] - tgt)
        num_acc = num_acc + jnp.sum(jnp.where(sel, rsum, f32(0.0)))
        den_acc = den_acc + jnp.sum(sel.astype(f32))

    o_ref[...] = (num_acc / den_acc) * jnp.ones((1, 1), f32)


def kernel(x, depth, B_c, exp_negative_beta_b):
    B, C, H, W = x.shape
    N = H * W
    L = 128
    R = N // L
    xr = x.reshape(B, C, R, L)
    dr = depth.reshape(B, R, L)
    bc = jnp.zeros((8, L), jnp.float32).at[:C].set(
        jnp.broadcast_to(B_c.reshape(C, 1), (C, L))
    )
    enb = jnp.ones((8, L), jnp.float32).at[:C].set(
        jnp.broadcast_to(exp_negative_beta_b.reshape(C, 1), (C, L))
    )
    out = pl.pallas_call(
        _backscatter_body,
        out_shape=jax.ShapeDtypeStruct((1, 1), jnp.float32),
    )(xr, dr, bc, enb)
    return out[0, 0]


# speedup vs baseline: 1.1599x; 1.0104x over previous
import jax
import jax.numpy as jnp
from jax import lax
from jax.experimental import pallas as pl

_GROUPS = 10
_K = 500


def _lane_scalar(vec, lane_idx, lane_iota):
    return jnp.sum(jnp.where(lane_iota == lane_idx, vec, 0.0))


def _backscatter_body(x_ref, d_ref, bc_ref, enb_ref, o_ref):
    B, C, R, L = x_ref.shape
    N = R * L
    f32 = jnp.float32
    i32 = jnp.int32
    idx_bits = int(N - 1).bit_length()

    dall = d_ref[...]
    dmin = jnp.min(dall)
    dmax = jnp.max(dall)

    def two_sum(a, b):
        s = a + b
        v = s - a
        e = (a - (s - v)) + (b - v)
        return s, e

    def split(a):
        c = a * f32(4097.0)
        hi = c - (c - a)
        return hi, a - hi

    def two_prod(a, b):
        p = a * b
        ah, al = split(a)
        bh, bl = split(b)
        e = ((ah * bh - p) + ah * bl + al * bh) + al * bl
        return p, e

    lane = lax.broadcasted_iota(i32, (1, L), 1)
    g = f32(_GROUPS)
    dh, dl = two_sum(dmax, -dmin)
    q1 = dh / g
    p, pe = two_prod(q1, g)
    t, te = two_sum(dh, -p)
    r = t + ((te - pe) + dl)
    q2 = r / g
    s_hi, s_lo = two_sum(q1, q2)
    idxv = lane.astype(f32)
    ph, pe2 = two_prod(jnp.full((1, L), s_hi), idxv)
    plo = pe2 + s_lo * idxv
    th, te2 = two_sum(ph, jnp.full((1, L), dmin))
    iv = th + (te2 + plo)
    iv = jnp.where(lane == 0, f32(0.0), iv)
    iv = jnp.where(lane == _GROUPS, dmax, iv)
    intervals = [_lane_scalar(iv, j, lane) for j in range(_GROUPS + 1)]

    cnts = [jnp.int32(0) for _ in range(_GROUPS)]
    gmaps = []
    for b in range(B):
        db = d_ref[b]
        gt = jnp.zeros((R, L), i32)
        for j in range(_GROUPS + 1):
            gt = gt + (db > intervals[j]).astype(i32)
        gmap = gt - 1
        gmaps.append(gmap)
        for i in range(_GROUPS):
            cnts[i] = cnts[i] + jnp.sum((gmap == i).astype(i32))
    ks = []
    for i in range(_GROUPS):
        numpix = cnts[i].astype(f32) / f32(B)
        kf = jnp.minimum(jnp.ceil(numpix * f32(0.01)), f32(_K))
        ks.append(kf.astype(i32))

    lgrows = [jnp.log(enb_ref[c : c + 1, :]) for c in range(C)]
    bcrows = [bc_ref[c : c + 1, :] for c in range(C)]

    pix_idx = (
        lax.broadcasted_iota(i32, (R, L), 0) * L
        + lax.broadcasted_iota(i32, (R, L), 1)
    )

    num_acc = f32(0.0)
    den_acc = f32(0.0)

    for b in range(B):
        db = d_ref[b]
        bright = (x_ref[b, 0] + x_ref[b, 1] + x_ref[b, 2]) / f32(C)
        bbits = lax.bitcast_convert_type(bright, i32)
        mbits = lax.bitcast_convert_type(bright * f32(1000.0), i32)
        gmap = gmaps[b]
        v = [
            jnp.where(gmap == i, bbits, mbits) for i in range(_GROUPS)
        ]

        def p1_body(it, carry):
            ts, cs = carry
            bitval = jnp.left_shift(jnp.int32(1), 30 - it)
            new_t, new_c = [], []
            for i in range(_GROUPS):
                cand = ts[i] + bitval
                cnt = jnp.sum((v[i] < cand).astype(i32))
                keep = cnt < ks[i]
                new_t.append(jnp.where(keep, cand, ts[i]))
                new_c.append(jnp.where(keep, cnt, cs[i]))
            return tuple(new_t), tuple(new_c)

        ts, c1 = lax.fori_loop(
            0,
            31,
            p1_body,
            (
                tuple(jnp.int32(0) for _ in range(_GROUPS)),
                tuple(jnp.int32(0) for _ in range(_GROUPS)),
            ),
        )

        tie = [v[i] == ts[i] for i in range(_GROUPS)]

        need_multi = jnp.bool_(False)
        for i in range(_GROUPS):
            need_multi = need_multi | ((ks[i] - c1[i] >= 2) & (ks[i] > 0))

        def p2_easy(_):
            return tuple(
                jnp.min(jnp.where(tie[i], pix_idx, jnp.int32(1 << 30)))
                for i in range(_GROUPS)
            )

        def p2_hard(_):
            def p2_body(it, js):
                bitval = jnp.left_shift(jnp.int32(1), idx_bits - 1 - it)
                new = []
                for i in range(_GROUPS):
                    cand = js[i] + bitval
                    cnt2 = jnp.sum((tie[i] & (pix_idx < cand)).astype(i32))
                    new.append(jnp.where(c1[i] + cnt2 < ks[i], cand, js[i]))
                return tuple(new)

            return lax.fori_loop(
                0, idx_bits, p2_body,
                tuple(jnp.int32(0) for _ in range(_GROUPS)),
            )

        js = lax.cond(need_multi, p2_hard, p2_easy, None)

        sel = jnp.zeros((R, L), jnp.bool_)
        for i in range(_GROUPS):
            si = (v[i] < ts[i]) | (tie[i] & (pix_idx <= js[i]))
            si = si & (ks[i] > 0)
            sel = sel | si

        rsum = jnp.zeros((R, L), f32)
        for c in range(C):
            tgt = bcrows[c] * (f32(1.0) - jnp.exp(db * lgrows[c]))
            rsum = rsum + jnp.abs(x_ref[b, c] - tgt)
        num_acc = num_acc + jnp.sum(jnp.where(sel, rsum, f32(0.0)))
        den_acc = den_acc + jnp.sum(sel.astype(f32))

    o_ref[...] = (num_acc / den_acc) * jnp.ones((1, 1), f32)


def kernel(x, depth, B_c, exp_negative_beta_b):
    B, C, H, W = x.shape
    N = H * W
    L = 128
    R = N // L
    xr = x.reshape(B, C, R, L)
    dr = depth.reshape(B, R, L)
    bc = jnp.zeros((8, L), jnp.float32).at[:C].set(
        jnp.broadcast_to(B_c.reshape(C, 1), (C, L))
    )
    enb = jnp.ones((8, L), jnp.float32).at[:C].set(
        jnp.broadcast_to(exp_negative_beta_b.reshape(C, 1), (C, L))
    )
    out = pl.pallas_call(
        _backscatter_body,
        out_shape=jax.ShapeDtypeStruct((1, 1), jnp.float32),
    )(xr, dr, bc, enb)
    return out[0, 0]
